# baseline (speedup 1.0000x reference)
import functools

import jax
import jax.numpy as jnp
from jax import lax
from jax.experimental import pallas as pl
from jax.experimental.pallas import tpu as pltpu
from jax.experimental.pallas import tpu_sc as plsc

BATCH = 16384
FIELDS = 26
D = 64
W = 128
B = BATCH * FIELDS
NW = 32
BPW = B // NW
CH = 832
NCH = BPW // CH


def _build():
    mesh = plsc.VectorSubcoreMesh(core_axis_name="c", subcore_axis_name="s")

    @functools.partial(
        pl.kernel,
        mesh=mesh,
        out_type=jax.ShapeDtypeStruct((B, W), jnp.float32),
        scratch_types=[
            pltpu.VMEM((NCH, CH), jnp.int32),
            pltpu.VMEM((2, CH, D), jnp.float32),
            pltpu.SemaphoreType.DMA,
            pltpu.SemaphoreType.DMA,
        ],
        compiler_params=pltpu.CompilerParams(use_tc_tiling_on_sc=False),
    )
    def emb_kernel(idx_hbm, table_hbm, out_hbm, idx_v, rows_v, sem0, sem1):
        sems = (sem0, sem1)
        wid = lax.axis_index("s") * 2 + lax.axis_index("c")
        base = wid * BPW
        pltpu.sync_copy(idx_hbm.at[wid], idx_v)

        pltpu.async_copy(table_hbm.at[idx_v.at[0]], rows_v.at[0], sem0)
        pltpu.async_copy(table_hbm.at[idx_v.at[1]], rows_v.at[1], sem1)

        def group(g, carry):
            for b in (0, 1):
                j = 2 * g + b
                pltpu.make_async_copy(
                    table_hbm.at[idx_v.at[0]], rows_v.at[b], sems[b]
                ).wait()
                pltpu.sync_copy(
                    rows_v.at[b],
                    out_hbm.at[pl.ds(base + j * CH, CH), pl.ds(0, D)],
                )
                nxt = jnp.minimum(j + 2, NCH - 1)
                pltpu.async_copy(table_hbm.at[idx_v.at[nxt]], rows_v.at[b], sems[b])
            return carry

        lax.fori_loop(0, NCH // 2, group, 0)
        pltpu.make_async_copy(table_hbm.at[idx_v.at[0]], rows_v.at[0], sem0).wait()
        pltpu.make_async_copy(table_hbm.at[idx_v.at[0]], rows_v.at[1], sem1).wait()

    return emb_kernel


_emb = _build()


@jax.jit
def kernel(token_ids, weight):
    idxf = token_ids.T.reshape(NW, NCH, CH).astype(jnp.int32)
    out2 = _emb(idxf, weight)
    out3 = out2.reshape(FIELDS, BATCH, W)[:, :, :D]
    return out3.transpose(1, 0, 2)


# speedup vs baseline: 1.0079x; 1.0079x over previous
import functools

import jax
import jax.numpy as jnp
from jax import lax
from jax.experimental import pallas as pl
from jax.experimental.pallas import tpu as pltpu
from jax.experimental.pallas import tpu_sc as plsc

BATCH = 16384
FIELDS = 26
D = 64
W = 128
B = BATCH * FIELDS
NW = 32
BPW = B // NW
CH = 512
NCH = BPW // CH


def _build():
    mesh = plsc.VectorSubcoreMesh(core_axis_name="c", subcore_axis_name="s")

    @functools.partial(
        pl.kernel,
        mesh=mesh,
        out_type=jax.ShapeDtypeStruct((B, W), jnp.float32),
        scratch_types=[
            pltpu.VMEM((NCH, CH), jnp.int32),
            pltpu.VMEM((2, CH, D), jnp.float32),
            pltpu.SemaphoreType.DMA,
            pltpu.SemaphoreType.DMA,
        ],
        compiler_params=pltpu.CompilerParams(use_tc_tiling_on_sc=False),
    )
    def emb_kernel(idx_hbm, table_hbm, out_hbm, idx_v, rows_v, sem0, sem1):
        sems = (sem0, sem1)
        wid = lax.axis_index("s") * 2 + lax.axis_index("c")
        base = wid * BPW
        pltpu.sync_copy(idx_hbm.at[wid], idx_v)

        pltpu.async_copy(table_hbm.at[idx_v.at[0]], rows_v.at[0], sem0)
        pltpu.async_copy(table_hbm.at[idx_v.at[1]], rows_v.at[1], sem1)

        def group(g, carry):
            for b in (0, 1):
                j = 2 * g + b
                pltpu.make_async_copy(
                    table_hbm.at[idx_v.at[0]], rows_v.at[b], sems[b]
                ).wait()
                pltpu.sync_copy(
                    rows_v.at[b],
                    out_hbm.at[pl.ds(base + j * CH, CH), pl.ds(0, D)],
                )
                nxt = jnp.minimum(j + 2, NCH - 1)
                pltpu.async_copy(table_hbm.at[idx_v.at[nxt]], rows_v.at[b], sems[b])
            return carry

        lax.fori_loop(0, NCH // 2, group, 0)
        pltpu.make_async_copy(table_hbm.at[idx_v.at[0]], rows_v.at[0], sem0).wait()
        pltpu.make_async_copy(table_hbm.at[idx_v.at[0]], rows_v.at[1], sem1).wait()

    return emb_kernel


_emb = _build()


@jax.jit
def kernel(token_ids, weight):
    idxf = token_ids.T.reshape(NW, NCH, CH).astype(jnp.int32)
    out2 = _emb(idxf, weight)
    out3 = out2.reshape(FIELDS, BATCH, W)[:, :, :D]
    return out3.transpose(1, 0, 2)
